# Initial kernel scaffold; baseline (speedup 1.0000x reference)
import jax
import jax.numpy as jnp
from jax.experimental import pallas as pl


def kernel(query_emb, slot_values, slot_keys, tids, centroid_codebook, slot_tids):
    raise NotImplementedError("write your pallas kernel here")


# speedup vs baseline: 2.0138x; 2.0138x over previous
import functools

import jax
import jax.numpy as jnp
from jax.experimental import pallas as pl
from jax.experimental.pallas import tpu as pltpu

N_BUCKETS = 512
S = 32
TAU = 0.1
ALPHA = 0.5


def _token_kernel(buckets_ref, tids_ref,
                  q_ref,
                  keys_ref,
                  vals_ref,
                  stids_ref,
                  cb_ref,
                  out_ref,
                  sim_ref):
    i = pl.program_id(0)
    bucket = buckets_ref[i]
    tid = tids_ref[i]

    q = q_ref[0, 0]
    qn = q * jax.lax.rsqrt(jnp.maximum(jnp.sum(q * q), 1e-24))
    anchor = cb_ref[pl.ds(bucket, 1), :]
    uq = ALPHA * qn + (1.0 - ALPHA) * anchor
    uq = uq * jax.lax.rsqrt(jnp.maximum(jnp.sum(uq * uq), 1e-24))

    keys = keys_ref[0]
    vals = vals_ref[0]
    scores = jax.lax.dot_general(
        uq, keys, (((1,), (1,)), ((), ())),
        preferred_element_type=jnp.float32)

    stids = stids_ref[0, 0]
    mask = (stids == tid).astype(jnp.float32)
    msum = jnp.sum(mask)
    has_match = msum > 0.0

    probs_hard = mask / (msum + 1e-9)
    s2 = scores * (1.0 / TAU)
    e = jnp.exp(s2 - jnp.max(s2))
    probs_soft = e / jnp.sum(e)
    probs = jnp.where(has_match, probs_hard, probs_soft)

    val = jax.lax.dot_general(
        probs, vals, (((1,), (0,)), ((), ())),
        preferred_element_type=jnp.float32)
    out_ref[0, 0] = val

    sim = jnp.where(has_match, 10.0, jnp.max(scores))
    sim_ref[0, 0, 0] = jnp.full((128,), sim, dtype=jnp.float32)


@jax.jit
def kernel(query_emb, slot_values, slot_keys, tids, centroid_codebook,
           slot_tids):
    B, T, D = query_emb.shape
    buckets = (tids % N_BUCKETS).reshape(B * T)
    tids_flat = tids.reshape(B * T)
    stids4 = slot_tids.reshape(B, N_BUCKETS, 1, S)
    q4 = query_emb.reshape(B, T, 1, D)

    grid = (B * T,)

    def q_map(i, bk, tf):
        return (i // T, i % T, 0, 0)

    def kv_map(i, bk, tf):
        return (i // T, bk[i], 0)

    def st_map(i, bk, tf):
        return (i // T, bk[i], 0, 0)

    def cb_map(i, bk, tf):
        return (0, 0)

    def out_map(i, bk, tf):
        return (i // T, i % T, 0, 0)

    grid_spec = pltpu.PrefetchScalarGridSpec(
        num_scalar_prefetch=2,
        grid=grid,
        in_specs=[
            pl.BlockSpec((1, 1, 1, D), q_map),
            pl.BlockSpec((1, S, D), kv_map),
            pl.BlockSpec((1, S, D), kv_map),
            pl.BlockSpec((1, 1, 1, S), st_map),
            pl.BlockSpec((N_BUCKETS, D), cb_map),
        ],
        out_specs=[
            pl.BlockSpec((1, 1, 1, D), out_map),
            pl.BlockSpec((1, 1, 1, 128), out_map),
        ],
    )

    out, sim = pl.pallas_call(
        _token_kernel,
        grid_spec=grid_spec,
        out_shape=[
            jax.ShapeDtypeStruct((B, T, 1, D), jnp.float32),
            jax.ShapeDtypeStruct((B, T, 1, 128), jnp.float32),
        ],
    )(buckets, tids_flat, q4, slot_keys, slot_values, stids4,
      centroid_codebook)
    return out.reshape(B, T, D), sim[:, :, 0, 0]


# speedup vs baseline: 3.5967x; 1.7860x over previous
import jax
import jax.numpy as jnp
from jax.experimental import pallas as pl
from jax.experimental.pallas import tpu as pltpu

N_BUCKETS = 512
S = 32
TAU = 0.1
ALPHA = 0.5
TB = 8


def _token_kernel(buckets_ref, tids_ref,
                  q_ref,
                  *refs):
    k_refs = refs[0:TB]
    v_refs = refs[TB:2 * TB]
    st_refs = refs[2 * TB:3 * TB]
    cb_ref = refs[3 * TB]
    out_ref = refs[3 * TB + 1]
    sim_ref = refs[3 * TB + 2]

    i = pl.program_id(0)
    base = i * TB

    qs = q_ref[0, 0]
    qn = qs * jax.lax.rsqrt(
        jnp.maximum(jnp.sum(qs * qs, axis=1, keepdims=True), 1e-24))

    for j in range(TB):
        bucket = buckets_ref[base + j]
        tid = tids_ref[base + j]
        anchor = cb_ref[pl.ds(bucket, 1), :]
        uq = ALPHA * qn[j:j + 1, :] + (1.0 - ALPHA) * anchor
        uq = uq * jax.lax.rsqrt(jnp.maximum(jnp.sum(uq * uq), 1e-24))

        keys = k_refs[j][0]
        vals = v_refs[j][0]
        scores = jax.lax.dot_general(
            uq, keys, (((1,), (1,)), ((), ())),
            preferred_element_type=jnp.float32)

        stids = st_refs[j][0, 0]
        mask = (stids == tid).astype(jnp.float32)
        msum = jnp.sum(mask)
        has_match = msum > 0.0

        probs_hard = mask / (msum + 1e-9)
        s2 = scores * (1.0 / TAU)
        e = jnp.exp(s2 - jnp.max(s2))
        probs_soft = e / jnp.sum(e)
        probs = jnp.where(has_match, probs_hard, probs_soft)

        val = jax.lax.dot_general(
            probs, vals, (((1,), (0,)), ((), ())),
            preferred_element_type=jnp.float32)
        out_ref[0, 0, j] = val[0]

        sim = jnp.where(has_match, 10.0, jnp.max(scores))
        sim_ref[0, 0, j] = jnp.full((128,), sim, dtype=jnp.float32)


@jax.jit
def kernel(query_emb, slot_values, slot_keys, tids, centroid_codebook,
           slot_tids):
    B, T, D = query_emb.shape
    G = T // TB
    buckets = (tids % N_BUCKETS).reshape(B * T)
    tids_flat = tids.reshape(B * T)
    stids4 = slot_tids.reshape(B, N_BUCKETS, 1, S)
    q4 = query_emb.reshape(B, G, TB, D)

    grid = (B * G,)

    def q_map(i, bk, tf):
        return (i // G, i % G, 0, 0)

    def kv_map(j):
        def m(i, bk, tf):
            return (i // G, bk[i * TB + j], 0)
        return m

    def st_map(j):
        def m(i, bk, tf):
            return (i // G, bk[i * TB + j], 0, 0)
        return m

    def cb_map(i, bk, tf):
        return (0, 0)

    def out_map(i, bk, tf):
        return (i // G, i % G, 0, 0)

    in_specs = [pl.BlockSpec((1, 1, TB, D), q_map)]
    in_specs += [pl.BlockSpec((1, S, D), kv_map(j)) for j in range(TB)]
    in_specs += [pl.BlockSpec((1, S, D), kv_map(j)) for j in range(TB)]
    in_specs += [pl.BlockSpec((1, 1, 1, S), st_map(j)) for j in range(TB)]
    in_specs += [pl.BlockSpec((N_BUCKETS, D), cb_map)]

    grid_spec = pltpu.PrefetchScalarGridSpec(
        num_scalar_prefetch=2,
        grid=grid,
        in_specs=in_specs,
        out_specs=[
            pl.BlockSpec((1, 1, TB, D), out_map),
            pl.BlockSpec((1, 1, TB, 128), out_map),
        ],
    )

    args = ([buckets, tids_flat, q4]
            + [slot_keys] * TB + [slot_values] * TB + [stids4] * TB
            + [centroid_codebook])
    out, sim = pl.pallas_call(
        _token_kernel,
        grid_spec=grid_spec,
        out_shape=[
            jax.ShapeDtypeStruct((B, G, TB, D), jnp.float32),
            jax.ShapeDtypeStruct((B, G, TB, 128), jnp.float32),
        ],
    )(*args)
    return out.reshape(B, T, D), sim[:, :, :, 0].reshape(B, T)


# speedup vs baseline: 3.7562x; 1.0443x over previous
import jax
import jax.numpy as jnp
from jax.experimental import pallas as pl
from jax.experimental.pallas import tpu as pltpu

N_BUCKETS = 512
S = 32
TAU = 0.1
ALPHA = 0.5
TB = 16


def _token_kernel(buckets_ref, tids_ref,
                  q_ref,
                  *refs):
    k_refs = refs[0:TB]
    v_refs = refs[TB:2 * TB]
    st_refs = refs[2 * TB:3 * TB]
    cb_ref = refs[3 * TB]
    out_ref = refs[3 * TB + 1]
    sim_ref = refs[3 * TB + 2]

    i = pl.program_id(0)
    base = i * TB

    qs = q_ref[0, 0]
    qn = qs * jax.lax.rsqrt(
        jnp.maximum(jnp.sum(qs * qs, axis=1, keepdims=True), 1e-24))

    for j in range(TB):
        bucket = buckets_ref[base + j]
        tid = tids_ref[base + j]
        anchor = cb_ref[pl.ds(bucket, 1), :]
        uq = ALPHA * qn[j:j + 1, :] + (1.0 - ALPHA) * anchor
        uq = uq * jax.lax.rsqrt(jnp.maximum(jnp.sum(uq * uq), 1e-24))

        keys = k_refs[j][0]
        vals = v_refs[j][0]
        scores = jax.lax.dot_general(
            uq, keys, (((1,), (1,)), ((), ())),
            preferred_element_type=jnp.float32)

        stids = st_refs[j][0, 0]
        mask = (stids == tid).astype(jnp.float32)
        msum = jnp.sum(mask)
        has_match = msum > 0.0

        probs_hard = mask / (msum + 1e-9)
        s2 = scores * (1.0 / TAU)
        e = jnp.exp(s2 - jnp.max(s2))
        probs_soft = e / jnp.sum(e)
        probs = jnp.where(has_match, probs_hard, probs_soft)

        val = jax.lax.dot_general(
            probs, vals, (((1,), (0,)), ((), ())),
            preferred_element_type=jnp.float32)
        out_ref[0, 0, j] = val[0]

        sim = jnp.where(has_match, 10.0, jnp.max(scores))
        sim_ref[0, 0, j] = jnp.full((128,), sim, dtype=jnp.float32)


@jax.jit
def kernel(query_emb, slot_values, slot_keys, tids, centroid_codebook,
           slot_tids):
    B, T, D = query_emb.shape
    G = T // TB
    buckets = (tids % N_BUCKETS).reshape(B * T)
    tids_flat = tids.reshape(B * T)
    stids4 = slot_tids.reshape(B, N_BUCKETS, 1, S)
    q4 = query_emb.reshape(B, G, TB, D)

    grid = (B * G,)

    def q_map(i, bk, tf):
        return (i // G, i % G, 0, 0)

    def kv_map(j):
        def m(i, bk, tf):
            return (i // G, bk[i * TB + j], 0)
        return m

    def st_map(j):
        def m(i, bk, tf):
            return (i // G, bk[i * TB + j], 0, 0)
        return m

    def cb_map(i, bk, tf):
        return (0, 0)

    def out_map(i, bk, tf):
        return (i // G, i % G, 0, 0)

    in_specs = [pl.BlockSpec((1, 1, TB, D), q_map)]
    in_specs += [pl.BlockSpec((1, S, D), kv_map(j)) for j in range(TB)]
    in_specs += [pl.BlockSpec((1, S, D), kv_map(j)) for j in range(TB)]
    in_specs += [pl.BlockSpec((1, 1, 1, S), st_map(j)) for j in range(TB)]
    in_specs += [pl.BlockSpec((N_BUCKETS, D), cb_map)]

    grid_spec = pltpu.PrefetchScalarGridSpec(
        num_scalar_prefetch=2,
        grid=grid,
        in_specs=in_specs,
        out_specs=[
            pl.BlockSpec((1, 1, TB, D), out_map),
            pl.BlockSpec((1, 1, TB, 128), out_map),
        ],
    )

    args = ([buckets, tids_flat, q4]
            + [slot_keys] * TB + [slot_values] * TB + [stids4] * TB
            + [centroid_codebook])
    out, sim = pl.pallas_call(
        _token_kernel,
        grid_spec=grid_spec,
        out_shape=[
            jax.ShapeDtypeStruct((B, G, TB, D), jnp.float32),
            jax.ShapeDtypeStruct((B, G, TB, 128), jnp.float32),
        ],
    )(*args)
    return out.reshape(B, T, D), sim[:, :, :, 0].reshape(B, T)
